# Initial kernel scaffold; baseline (speedup 1.0000x reference)
import jax
import jax.numpy as jnp
from jax.experimental import pallas as pl


def kernel(input_tensor, colors):
    raise NotImplementedError("write your pallas kernel here")


# speedup vs baseline: 1287.2932x; 1287.2932x over previous
import functools

import jax
import jax.numpy as jnp
from jax import lax
from jax.experimental import pallas as pl
from jax.experimental.pallas import tpu as pltpu
from jax.experimental.pallas import tpu_sc as plsc

_B, _H, _W = 16, 512, 512
_HW = _H * _W
_N = _B * _HW
_NC, _NS, _L = 2, 16, 16
_NW = _NC * _NS
_PW = _N // _NW
_IMGS_PER_W = _HW // _PW
_C = 8192
_CHUNKS = _PW // _C
_TBL = 256


def _sc_colormap(x_flat, colors_flat):
    mesh = plsc.VectorSubcoreMesh(core_axis_name="c", subcore_axis_name="s")

    @functools.partial(
        pl.kernel,
        out_type=jax.ShapeDtypeStruct((_N * 3,), jnp.float32),
        mesh=mesh,
        compiler_params=pltpu.CompilerParams(needs_layout_passes=False),
        scratch_types=[
            pltpu.VMEM((3 * _TBL,), jnp.float32),
            pltpu.VMEM((_C,), jnp.int32),
            pltpu.VMEM((3 * _C,), jnp.float32),
        ],
    )
    def run(x_hbm, colors_hbm, out_hbm, tbl_v, idx_v, ob_v):
        wid = lax.axis_index("s") * _NC + lax.axis_index("c")
        pltpu.sync_copy(colors_hbm, tbl_v)
        b = wid // _IMGS_PER_W
        half = wid % _IMGS_PER_W
        in_base = wid * _PW

        def chunk_body(j, carry):
            pltpu.sync_copy(x_hbm.at[pl.ds(in_base + j * _C, _C)], idx_v)

            def body(i, c2):
                raw = idx_v[pl.ds(i * _L, _L)]
                idx = jnp.clip(raw, 0, _TBL - 1)
                for c in range(3):
                    vals = plsc.load_gather(tbl_v, [idx + (c * _TBL)])
                    ob_v[pl.ds(c * _C + i * _L, _L)] = vals
                return c2

            lax.fori_loop(0, _C // _L, body, 0)
            for c in range(3):
                out_off = (b * 3 + c) * _HW + half * _PW + j * _C
                pltpu.sync_copy(ob_v.at[pl.ds(c * _C, _C)],
                                out_hbm.at[pl.ds(out_off, _C)])
            return carry

        lax.fori_loop(0, _CHUNKS, chunk_body, 0)

    return run(x_flat, colors_flat)


def kernel(input_tensor, colors):
    x_flat = input_tensor.reshape(_N)
    colors_flat = colors.reshape(3 * _TBL)
    out = _sc_colormap(x_flat, colors_flat)
    return out.reshape(_B, 3, _H, _W)


# speedup vs baseline: 2480.5008x; 1.9269x over previous
import functools

import jax
import jax.numpy as jnp
from jax import lax
from jax.experimental import pallas as pl
from jax.experimental.pallas import tpu as pltpu
from jax.experimental.pallas import tpu_sc as plsc

_B, _H, _W = 16, 512, 512
_HW = _H * _W
_N = _B * _HW
_NC, _NS, _L = 2, 16, 16
_NW = _NC * _NS
_PW = _N // _NW
_IMGS_PER_W = _HW // _PW
_C = 8192
_CHUNKS = _PW // _C
_TBL = 256


def _sc_colormap(x_flat, colors_flat):
    mesh = plsc.VectorSubcoreMesh(core_axis_name="c", subcore_axis_name="s")

    @functools.partial(
        pl.kernel,
        out_type=jax.ShapeDtypeStruct((_N * 3,), jnp.float32),
        mesh=mesh,
        compiler_params=pltpu.CompilerParams(needs_layout_passes=False),
        scratch_types=[
            pltpu.VMEM((3 * _TBL,), jnp.float32),
            pltpu.VMEM((_C,), jnp.int32),
            pltpu.VMEM((3 * _C,), jnp.float32),
        ],
    )
    def run(x_hbm, colors_hbm, out_hbm, tbl_v, idx_v, ob_v):
        wid = lax.axis_index("s") * _NC + lax.axis_index("c")
        pltpu.sync_copy(colors_hbm, tbl_v)
        b = wid // _IMGS_PER_W
        half = wid % _IMGS_PER_W
        in_base = wid * _PW

        def chunk_body(j, carry):
            pltpu.sync_copy(x_hbm.at[pl.ds(in_base + j * _C, _C)], idx_v)

            @plsc.parallel_loop(0, _C // _L, 1, unroll=8)
            def body(i):
                raw = idx_v[pl.ds(i * _L, _L)]
                idx = jnp.clip(raw, 0, _TBL - 1)
                for c in range(3):
                    vals = plsc.load_gather(tbl_v, [idx + (c * _TBL)])
                    ob_v[pl.ds(c * _C + i * _L, _L)] = vals
            for c in range(3):
                out_off = (b * 3 + c) * _HW + half * _PW + j * _C
                pltpu.sync_copy(ob_v.at[pl.ds(c * _C, _C)],
                                out_hbm.at[pl.ds(out_off, _C)])
            return carry

        lax.fori_loop(0, _CHUNKS, chunk_body, 0)

    return run(x_flat, colors_flat)


def kernel(input_tensor, colors):
    x_flat = input_tensor.reshape(_N)
    colors_flat = colors.reshape(3 * _TBL)
    out = _sc_colormap(x_flat, colors_flat)
    return out.reshape(_B, 3, _H, _W)


# speedup vs baseline: 3016.5310x; 1.2161x over previous
import functools

import jax
import jax.numpy as jnp
from jax import lax
from jax.experimental import pallas as pl
from jax.experimental.pallas import tpu as pltpu
from jax.experimental.pallas import tpu_sc as plsc

_B, _H, _W = 16, 512, 512
_HW = _H * _W
_N = _B * _HW
_NC, _NS, _L = 2, 16, 16
_NW = _NC * _NS
_PW = _N // _NW
_IMGS_PER_W = _HW // _PW
_C = 8192
_CHUNKS = _PW // _C
_TBL = 256


def _sc_colormap(x_flat, colors_flat):
    mesh = plsc.VectorSubcoreMesh(core_axis_name="c", subcore_axis_name="s")

    @functools.partial(
        pl.kernel,
        out_type=jax.ShapeDtypeStruct((_N * 3,), jnp.float32),
        mesh=mesh,
        compiler_params=pltpu.CompilerParams(needs_layout_passes=False),
        scratch_types=[
            pltpu.VMEM((3 * _TBL,), jnp.float32),
            pltpu.VMEM((2 * _C,), jnp.int32),
            pltpu.VMEM((2 * 3 * _C,), jnp.float32),
            pltpu.SemaphoreType.DMA,
            pltpu.SemaphoreType.DMA,
            pltpu.SemaphoreType.DMA,
            pltpu.SemaphoreType.DMA,
        ],
    )
    def run(x_hbm, colors_hbm, out_hbm, tbl_v, idx_v, ob_v,
            sin0, sin1, sout0, sout1):
        wid = lax.axis_index("s") * _NC + lax.axis_index("c")
        pltpu.sync_copy(colors_hbm, tbl_v)
        b = wid // _IMGS_PER_W
        half = wid % _IMGS_PER_W
        in_base = wid * _PW
        sins = (sin0, sin1)
        souts = (sout0, sout1)
        in_handles = [None, None]
        out_handles = [[], []]

        in_handles[0] = pltpu.async_copy(
            x_hbm.at[pl.ds(in_base, _C)], idx_v.at[pl.ds(0, _C)], sins[0])
        for j in range(_CHUNKS):
            s = j % 2
            if j + 1 < _CHUNKS:
                ns = (j + 1) % 2
                in_handles[ns] = pltpu.async_copy(
                    x_hbm.at[pl.ds(in_base + (j + 1) * _C, _C)],
                    idx_v.at[pl.ds(ns * _C, _C)], sins[ns])
            in_handles[s].wait()
            for h in out_handles[s]:
                h.wait()
            out_handles[s] = []

            @plsc.parallel_loop(0, _C // _L, 1, unroll=8)
            def body(i, s=s):
                raw = idx_v[pl.ds(s * _C + i * _L, _L)]
                idx = jnp.clip(raw, 0, _TBL - 1)
                for c in range(3):
                    vals = plsc.load_gather(tbl_v, [idx + (c * _TBL)])
                    ob_v[pl.ds((s * 3 + c) * _C + i * _L, _L)] = vals

            for c in range(3):
                out_off = (b * 3 + c) * _HW + half * _PW + j * _C
                out_handles[s].append(pltpu.async_copy(
                    ob_v.at[pl.ds((s * 3 + c) * _C, _C)],
                    out_hbm.at[pl.ds(out_off, _C)], souts[s]))
        for s in range(2):
            for h in out_handles[s]:
                h.wait()

    return run(x_flat, colors_flat)


def kernel(input_tensor, colors):
    x_flat = input_tensor.reshape(_N)
    colors_flat = colors.reshape(3 * _TBL)
    out = _sc_colormap(x_flat, colors_flat)
    return out.reshape(_B, 3, _H, _W)


# speedup vs baseline: 7107.8863x; 2.3563x over previous
import functools

import jax
import jax.numpy as jnp
from jax import lax
from jax.experimental import pallas as pl
from jax.experimental.pallas import tpu as pltpu
from jax.experimental.pallas import tpu_sc as plsc

_B, _H, _W = 16, 512, 512
_NC, _NS, _L = 2, 16, 16
_NW = _NC * _NS
_RW = _H // 2
_CR = 16
_C = _CR * _W
_CHUNKS = _RW // _CR
_TBL = 256


def _sc_colormap(x, colors_flat):
    mesh = plsc.VectorSubcoreMesh(core_axis_name="c", subcore_axis_name="s")

    @functools.partial(
        pl.kernel,
        out_type=jax.ShapeDtypeStruct((_B, 3, _H, _W), jnp.float32),
        mesh=mesh,
        compiler_params=pltpu.CompilerParams(
            needs_layout_passes=False, use_tc_tiling_on_sc=True),
        scratch_types=[
            pltpu.VMEM((3 * _TBL,), jnp.float32),
            pltpu.VMEM((2 * _CR, _W), jnp.int32),
            pltpu.VMEM((2 * 3 * _CR, _W), jnp.float32),
            pltpu.SemaphoreType.DMA,
            pltpu.SemaphoreType.DMA,
            pltpu.SemaphoreType.DMA,
            pltpu.SemaphoreType.DMA,
        ],
    )
    def run(x_hbm, colors_hbm, out_hbm, tbl_v, idx_v, ob_v,
            sin0, sin1, sout0, sout1):
        wid = lax.axis_index("s") * _NC + lax.axis_index("c")
        pltpu.sync_copy(colors_hbm, tbl_v)
        b = wid // 2
        row_base = (wid % 2) * _RW
        sins = (sin0, sin1)
        souts = (sout0, sout1)
        in_handles = [None, None]
        out_handles = [[], []]

        in_handles[0] = pltpu.async_copy(
            x_hbm.at[b, 0, pl.ds(row_base, _CR), :],
            idx_v.at[pl.ds(0, _CR), :], sins[0])
        for j in range(_CHUNKS):
            s = j % 2
            if j + 1 < _CHUNKS:
                ns = (j + 1) % 2
                in_handles[ns] = pltpu.async_copy(
                    x_hbm.at[b, 0, pl.ds(row_base + (j + 1) * _CR, _CR), :],
                    idx_v.at[pl.ds(ns * _CR, _CR), :], sins[ns])
            in_handles[s].wait()
            for h in out_handles[s]:
                h.wait()
            out_handles[s] = []

            @plsc.parallel_loop(0, _C // _L, 1, unroll=8)
            def body(i, s=s):
                row = i >> 5
                col = (i & 31) * _L
                raw = idx_v[s * _CR + row, pl.ds(col, _L)]
                idx = jnp.clip(raw, 0, _TBL - 1)
                for c in range(3):
                    vals = plsc.load_gather(tbl_v, [idx + (c * _TBL)])
                    ob_v[(s * 3 + c) * _CR + row, pl.ds(col, _L)] = vals

            for c in range(3):
                out_handles[s].append(pltpu.async_copy(
                    ob_v.at[pl.ds((s * 3 + c) * _CR, _CR), :],
                    out_hbm.at[b, c, pl.ds(row_base + j * _CR, _CR), :],
                    souts[s]))
        for s in range(2):
            for h in out_handles[s]:
                h.wait()

    return run(x, colors_flat)


def kernel(input_tensor, colors):
    return _sc_colormap(input_tensor, colors.reshape(3 * _TBL))
